# baseline (speedup 1.0000x reference)
import functools

import jax
import jax.numpy as jnp
from jax.experimental import pallas as pl
from jax.experimental.pallas import tpu as pltpu

Q = 512
S = 20000
C = 18
THR = 0.75
K = 100
TILE = 2048
SALIGN = (S // 128) * 128
NSTEP = (SALIGN + TILE - 1) // TILE


def _body(masks_hbm, tail_ref, labels_ref, pscore_ref, out_ref,
          buf0, buf1, sem0, sem1):
    bufs = (buf0, buf1)
    sems = (sem0, sem1)
    widths = [TILE] * (NSTEP - 1) + [SALIGN - TILE * (NSTEP - 1)]

    def copy(j):
        return pltpu.make_async_copy(
            masks_hbm.at[:, pl.ds(j * TILE, widths[j])],
            bufs[j % 2].at[:, pl.ds(0, widths[j])],
            sems[j % 2])

    copy(0).start()
    inter = jnp.zeros((Q, Q), jnp.float32)
    sigrow = jnp.zeros((1, Q), jnp.float32)

    def accum(x, inter, sigrow):
        w = x.shape[1]
        pos = x > 0.0
        bin16 = jnp.where(pos, 1.0, 0.0).astype(jnp.bfloat16)
        inter = inter + jax.lax.dot_general(
            bin16, bin16, (((1,), (1,)), ((), ())),
            preferred_element_type=jnp.float32)
        sig = jnp.where(pos, 0.5 + 0.5 * jnp.tanh(0.5 * x), 0.0)
        sigrow = sigrow + jax.lax.dot_general(
            jnp.ones((1, w), jnp.float32), sig, (((1,), (1,)), ((), ())),
            precision=jax.lax.Precision.HIGHEST,
            preferred_element_type=jnp.float32)
        return inter, sigrow

    for j in range(NSTEP):
        if j + 1 < NSTEP:
            copy(j + 1).start()
        copy(j).wait()
        x = bufs[j % 2][:, :widths[j]]
        inter, sigrow = accum(x, inter, sigrow)
    inter, sigrow = accum(tail_ref[...], inter, sigrow)

    lt = labels_ref[...]
    m = jnp.max(lt, axis=0, keepdims=True)
    e = jnp.exp(lt - m)
    denom = jnp.sum(e, axis=0, keepdims=True)
    scores_t = (e[:C, :] / denom) * pscore_ref[...]

    ri = jax.lax.broadcasted_iota(jnp.int32, (Q, Q), 0)
    ci = jax.lax.broadcasted_iota(jnp.int32, (Q, Q), 1)
    eye = ri == ci
    pn_row = jnp.sum(jnp.where(eye, inter, 0.0), axis=0, keepdims=True)
    pn_col = jnp.sum(jnp.where(eye, inter, 0.0), axis=1, keepdims=True)

    nms_row = jnp.max(scores_t, axis=0, keepdims=True)
    nms_row = jnp.where(pn_row == 0.0, 0.0, nms_row)

    union = pn_col + pn_row - inter + 1e-6
    adj = jnp.where(inter > THR * union, 1.0, 0.0).astype(jnp.bfloat16)

    s_col = jnp.sum(jnp.where(eye, jnp.broadcast_to(nms_row, (Q, Q)), 0.0),
                    axis=1, keepdims=True)
    gt = jnp.where(nms_row > s_col, 1.0, 0.0)
    eqlt = jnp.where((nms_row == s_col) & (ci < ri), 1.0, 0.0)
    rank_col = jnp.sum(gt + eqlt, axis=1, keepdims=True)
    rank_row = jnp.sum(jnp.where(eye, jnp.broadcast_to(rank_col, (Q, Q)), 0.0),
                       axis=0, keepdims=True)
    perm16 = jnp.where(ri.astype(jnp.float32) == rank_row, 1.0, 0.0
                       ).astype(jnp.bfloat16)

    padj = jax.lax.dot_general(
        perm16, adj, (((1,), (0,)), ((), ())),
        preferred_element_type=jnp.float32)
    sadj = jax.lax.dot_general(
        padj.astype(jnp.bfloat16), perm16, (((1,), (1,)), ((), ())),
        preferred_element_type=jnp.float32)
    sadj16 = jnp.where(ci > ri, sadj, 0.0).astype(jnp.bfloat16)

    def nms_cond(carry):
        _, changed = carry
        return changed

    def nms_sweep(carry):
        keep, _ = carry
        supcnt = jax.lax.dot_general(
            keep.astype(jnp.bfloat16), sadj16, (((1,), (0,)), ((), ())),
            preferred_element_type=jnp.float32)
        keep_new = jnp.where(supcnt > 0.0, 0.0, 1.0)
        changed = jnp.sum(jnp.abs(keep_new - keep)) > 0.0
        return keep_new, changed

    keep, _ = jax.lax.while_loop(
        nms_cond, nms_sweep, (jnp.ones((1, Q), jnp.float32), True))

    keep_row = jax.lax.dot_general(
        keep.astype(jnp.bfloat16), perm16, (((1,), (0,)), ((), ())),
        preferred_element_type=jnp.float32)

    w_row = sigrow / (pn_row + 1e-6)
    a = scores_t * keep_row

    wb = jnp.broadcast_to(w_row, (C, Q))
    lane128 = jax.lax.broadcasted_iota(jnp.int32, (1, 128), 1)

    acc = jnp.zeros((1, 128), jnp.float32)
    for k in range(K):
        v = jnp.max(a)
        hit = a == v
        wv = jnp.sum(jnp.where(hit, wb, 0.0))
        acc = jnp.where(lane128 == k, jnp.maximum(v, 0.0) * wv, acc)
        a = jnp.where(hit, -1.0, a)
    out_ref[...] = acc


@functools.partial(jax.jit, static_argnames=())
def kernel(pred_labels, pred_masks, pred_scores):
    labels_t = pred_labels.T
    pscore_row = pred_scores.reshape(1, Q)
    tail = jax.lax.slice(pred_masks, (0, SALIGN), (Q, S))
    out = pl.pallas_call(
        _body,
        in_specs=[
            pl.BlockSpec(memory_space=pltpu.MemorySpace.HBM),
            pl.BlockSpec((Q, S - SALIGN), lambda: (0, 0)),
            pl.BlockSpec((C + 1, Q), lambda: (0, 0)),
            pl.BlockSpec((1, Q), lambda: (0, 0)),
        ],
        out_specs=pl.BlockSpec((1, 128), lambda: (0, 0)),
        out_shape=jax.ShapeDtypeStruct((1, 128), jnp.float32),
        scratch_shapes=[
            pltpu.VMEM((Q, TILE), jnp.float32),
            pltpu.VMEM((Q, TILE), jnp.float32),
            pltpu.SemaphoreType.DMA,
            pltpu.SemaphoreType.DMA,
        ],
    )(pred_masks, tail, labels_t, pscore_row)
    return out[0, :K]


# speedup vs baseline: 1.0193x; 1.0193x over previous
import functools

import jax
import jax.numpy as jnp
from jax.experimental import pallas as pl
from jax.experimental.pallas import tpu as pltpu

Q = 512
S = 20000
C = 18
THR = 0.75
K = 100
TILE = 2048
SALIGN = (S // 128) * 128
NSTEP = (SALIGN + TILE - 1) // TILE


def _body(masks_hbm, tail_ref, labels_ref, pscore_ref, out_ref,
          buf0, buf1, sem0, sem1):
    bufs = (buf0, buf1)
    sems = (sem0, sem1)
    widths = [TILE] * (NSTEP - 1) + [SALIGN - TILE * (NSTEP - 1)]

    def copy(j):
        return pltpu.make_async_copy(
            masks_hbm.at[:, pl.ds(j * TILE, widths[j])],
            bufs[j % 2].at[:, pl.ds(0, widths[j])],
            sems[j % 2])

    copy(0).start()
    inter = jnp.zeros((Q, Q), jnp.float32)
    sigrow = jnp.zeros((1, Q), jnp.float32)

    def accum(x, inter, sigrow):
        w = x.shape[1]
        bin16 = jnp.where(x > 0.0, 1.0, 0.0).astype(jnp.bfloat16)
        inter = inter + jax.lax.dot_general(
            bin16, bin16, (((1,), (1,)), ((), ())),
            preferred_element_type=jnp.float32)
        th = jnp.maximum(jnp.tanh(0.5 * x), 0.0)
        sigrow = sigrow + jax.lax.dot_general(
            jnp.ones((1, w), jnp.float32), th, (((1,), (1,)), ((), ())),
            precision=jax.lax.Precision.HIGHEST,
            preferred_element_type=jnp.float32)
        return inter, sigrow

    for j in range(NSTEP):
        if j + 1 < NSTEP:
            copy(j + 1).start()
        copy(j).wait()
        x = bufs[j % 2][:, :widths[j]]
        inter, sigrow = accum(x, inter, sigrow)
    inter, sigrow = accum(tail_ref[...], inter, sigrow)

    lt = labels_ref[...]
    m = jnp.max(lt, axis=0, keepdims=True)
    e = jnp.exp(lt - m)
    denom = jnp.sum(e, axis=0, keepdims=True)
    scores_t = (e[:C, :] / denom) * pscore_ref[...]

    ri = jax.lax.broadcasted_iota(jnp.int32, (Q, Q), 0)
    ci = jax.lax.broadcasted_iota(jnp.int32, (Q, Q), 1)
    eye = ri == ci
    pn_row = jnp.sum(jnp.where(eye, inter, 0.0), axis=0, keepdims=True)
    pn_col = jnp.sum(jnp.where(eye, inter, 0.0), axis=1, keepdims=True)

    nms_row = jnp.max(scores_t, axis=0, keepdims=True)
    nms_row = jnp.where(pn_row == 0.0, 0.0, nms_row)

    union = pn_col + pn_row - inter + 1e-6
    adj = jnp.where(inter > THR * union, 1.0, 0.0).astype(jnp.bfloat16)

    s_col = jnp.sum(jnp.where(eye, jnp.broadcast_to(nms_row, (Q, Q)), 0.0),
                    axis=1, keepdims=True)
    gt = jnp.where(nms_row > s_col, 1.0, 0.0)
    eqlt = jnp.where((nms_row == s_col) & (ci < ri), 1.0, 0.0)
    rank_col = jnp.sum(gt + eqlt, axis=1, keepdims=True)
    rank_row = jnp.sum(jnp.where(eye, jnp.broadcast_to(rank_col, (Q, Q)), 0.0),
                       axis=0, keepdims=True)
    perm16 = jnp.where(ri.astype(jnp.float32) == rank_row, 1.0, 0.0
                       ).astype(jnp.bfloat16)

    padj = jax.lax.dot_general(
        perm16, adj, (((1,), (0,)), ((), ())),
        preferred_element_type=jnp.float32)
    sadj = jax.lax.dot_general(
        padj.astype(jnp.bfloat16), perm16, (((1,), (1,)), ((), ())),
        preferred_element_type=jnp.float32)
    sadj16 = jnp.where(ci > ri, sadj, 0.0).astype(jnp.bfloat16)

    def nms_cond(carry):
        _, changed = carry
        return changed

    def one_sweep(keep):
        supcnt = jax.lax.dot_general(
            keep.astype(jnp.bfloat16), sadj16, (((1,), (0,)), ((), ())),
            preferred_element_type=jnp.float32)
        return jnp.where(supcnt > 0.0, 0.0, 1.0)

    def nms_sweep(carry):
        keep, _ = carry
        keep_new = one_sweep(one_sweep(keep))
        changed = jnp.sum(jnp.abs(keep_new - keep)) > 0.0
        return keep_new, changed

    keep, _ = jax.lax.while_loop(
        nms_cond, nms_sweep, (jnp.ones((1, Q), jnp.float32), True))

    keep_row = jax.lax.dot_general(
        keep.astype(jnp.bfloat16), perm16, (((1,), (0,)), ((), ())),
        preferred_element_type=jnp.float32)

    w_row = (0.5 * pn_row + 0.5 * sigrow) / (pn_row + 1e-6)
    a = scores_t * keep_row

    wb = jnp.broadcast_to(w_row, (C, Q))
    lane128 = jax.lax.broadcasted_iota(jnp.int32, (1, 128), 1)

    acc = jnp.zeros((1, 128), jnp.float32)
    for k in range(K):
        v = jnp.max(a)
        hit = a == v
        wv = jnp.sum(jnp.where(hit, wb, 0.0))
        acc = jnp.where(lane128 == k, jnp.maximum(v, 0.0) * wv, acc)
        a = jnp.where(hit, -1.0, a)
    out_ref[...] = acc


@functools.partial(jax.jit, static_argnames=())
def kernel(pred_labels, pred_masks, pred_scores):
    labels_t = pred_labels.T
    pscore_row = pred_scores.reshape(1, Q)
    tail = jax.lax.slice(pred_masks, (0, SALIGN), (Q, S))
    out = pl.pallas_call(
        _body,
        in_specs=[
            pl.BlockSpec(memory_space=pltpu.MemorySpace.HBM),
            pl.BlockSpec((Q, S - SALIGN), lambda: (0, 0)),
            pl.BlockSpec((C + 1, Q), lambda: (0, 0)),
            pl.BlockSpec((1, Q), lambda: (0, 0)),
        ],
        out_specs=pl.BlockSpec((1, 128), lambda: (0, 0)),
        out_shape=jax.ShapeDtypeStruct((1, 128), jnp.float32),
        scratch_shapes=[
            pltpu.VMEM((Q, TILE), jnp.float32),
            pltpu.VMEM((Q, TILE), jnp.float32),
            pltpu.SemaphoreType.DMA,
            pltpu.SemaphoreType.DMA,
        ],
    )(pred_masks, tail, labels_t, pscore_row)
    return out[0, :K]


# speedup vs baseline: 1.3632x; 1.3374x over previous
import functools

import jax
import jax.numpy as jnp
from jax.experimental import pallas as pl
from jax.experimental.pallas import tpu as pltpu

Q = 512
S = 20000
C = 18
THR = 0.75
K = 100
TILE = 2048
SALIGN = (S // 128) * 128
NSTEP = (SALIGN + TILE - 1) // TILE


def _body(masks_hbm, tail_ref, labels_ref, pscore_ref, out_ref,
          buf0, buf1, sem0, sem1):
    bufs = (buf0, buf1)
    sems = (sem0, sem1)
    widths = [TILE] * (NSTEP - 1) + [SALIGN - TILE * (NSTEP - 1)]

    def copy(j):
        return pltpu.make_async_copy(
            masks_hbm.at[:, pl.ds(j * TILE, widths[j])],
            bufs[j % 2].at[:, pl.ds(0, widths[j])],
            sems[j % 2])

    copy(0).start()
    inter = jnp.zeros((Q, Q), jnp.float32)
    sigrow = jnp.zeros((1, Q), jnp.float32)

    def accum(x, inter, sigrow):
        w = x.shape[1]
        bin16 = jnp.where(x > 0.0, 1.0, 0.0).astype(jnp.bfloat16)
        inter = inter + jax.lax.dot_general(
            bin16, bin16, (((1,), (1,)), ((), ())),
            preferred_element_type=jnp.float32)
        th = jnp.maximum(jnp.tanh(0.5 * x), 0.0).astype(jnp.bfloat16)
        sigrow = sigrow + jax.lax.dot_general(
            jnp.ones((1, w), jnp.bfloat16), th, (((1,), (1,)), ((), ())),
            preferred_element_type=jnp.float32)
        return inter, sigrow

    for j in range(NSTEP):
        if j + 1 < NSTEP:
            copy(j + 1).start()
        copy(j).wait()
        x = bufs[j % 2][:, :widths[j]]
        inter, sigrow = accum(x, inter, sigrow)
    inter, sigrow = accum(tail_ref[...], inter, sigrow)

    lt = labels_ref[...]
    m = jnp.max(lt, axis=0, keepdims=True)
    e = jnp.exp(lt - m)
    denom = jnp.sum(e, axis=0, keepdims=True)
    scores_t = (e[:C, :] / denom) * pscore_ref[...]

    ri = jax.lax.broadcasted_iota(jnp.int32, (Q, Q), 0)
    ci = jax.lax.broadcasted_iota(jnp.int32, (Q, Q), 1)
    eye = ri == ci
    pn_row = jnp.sum(jnp.where(eye, inter, 0.0), axis=0, keepdims=True)
    pn_col = jnp.sum(jnp.where(eye, inter, 0.0), axis=1, keepdims=True)

    nms_row = jnp.max(scores_t, axis=0, keepdims=True)
    nms_row = jnp.where(pn_row == 0.0, 0.0, nms_row)

    union = pn_col + pn_row - inter + 1e-6
    adj = jnp.where(inter > THR * union, 1.0, 0.0).astype(jnp.bfloat16)

    s_col = jnp.sum(jnp.where(eye, jnp.broadcast_to(nms_row, (Q, Q)), 0.0),
                    axis=1, keepdims=True)
    gt = jnp.where(nms_row > s_col, 1.0, 0.0)
    eqlt = jnp.where((nms_row == s_col) & (ci < ri), 1.0, 0.0)
    rank_col = jnp.sum(gt + eqlt, axis=1, keepdims=True)
    rank_row = jnp.sum(jnp.where(eye, jnp.broadcast_to(rank_col, (Q, Q)), 0.0),
                       axis=0, keepdims=True)
    perm16 = jnp.where(ri.astype(jnp.float32) == rank_row, 1.0, 0.0
                       ).astype(jnp.bfloat16)

    padj = jax.lax.dot_general(
        perm16, adj, (((1,), (0,)), ((), ())),
        preferred_element_type=jnp.float32)
    sadj = jax.lax.dot_general(
        padj.astype(jnp.bfloat16), perm16, (((1,), (1,)), ((), ())),
        preferred_element_type=jnp.float32)
    sadj16 = jnp.where(ci > ri, sadj, 0.0).astype(jnp.bfloat16)

    def nms_cond(carry):
        _, changed = carry
        return changed

    def one_sweep(keep):
        supcnt = jax.lax.dot_general(
            keep.astype(jnp.bfloat16), sadj16, (((1,), (0,)), ((), ())),
            preferred_element_type=jnp.float32)
        return jnp.where(supcnt > 0.0, 0.0, 1.0)

    def nms_sweep(carry):
        keep, _ = carry
        keep_new = one_sweep(one_sweep(keep))
        changed = jnp.sum(jnp.abs(keep_new - keep)) > 0.0
        return keep_new, changed

    keep, _ = jax.lax.while_loop(
        nms_cond, nms_sweep, (jnp.ones((1, Q), jnp.float32), True))

    keep_row = jax.lax.dot_general(
        keep.astype(jnp.bfloat16), perm16, (((1,), (0,)), ((), ())),
        preferred_element_type=jnp.float32)

    w_row = (0.5 * pn_row + 0.5 * sigrow) / (pn_row + 1e-6)
    a = scores_t * keep_row

    wb = jnp.broadcast_to(w_row, (C, Q))
    lane128 = jax.lax.broadcasted_iota(jnp.int32, (1, 128), 1)

    acc = jnp.zeros((1, 128), jnp.float32)
    for k in range(K):
        v = jnp.max(a)
        hit = a == v
        wv = jnp.sum(jnp.where(hit, wb, 0.0))
        acc = jnp.where(lane128 == k, jnp.maximum(v, 0.0) * wv, acc)
        a = jnp.where(hit, -1.0, a)
    out_ref[...] = acc


@functools.partial(jax.jit, static_argnames=())
def kernel(pred_labels, pred_masks, pred_scores):
    labels_t = pred_labels.T
    pscore_row = pred_scores.reshape(1, Q)
    tail = jax.lax.slice(pred_masks, (0, SALIGN), (Q, S))
    out = pl.pallas_call(
        _body,
        in_specs=[
            pl.BlockSpec(memory_space=pltpu.MemorySpace.HBM),
            pl.BlockSpec((Q, S - SALIGN), lambda: (0, 0)),
            pl.BlockSpec((C + 1, Q), lambda: (0, 0)),
            pl.BlockSpec((1, Q), lambda: (0, 0)),
        ],
        out_specs=pl.BlockSpec((1, 128), lambda: (0, 0)),
        out_shape=jax.ShapeDtypeStruct((1, 128), jnp.float32),
        scratch_shapes=[
            pltpu.VMEM((Q, TILE), jnp.float32),
            pltpu.VMEM((Q, TILE), jnp.float32),
            pltpu.SemaphoreType.DMA,
            pltpu.SemaphoreType.DMA,
        ],
    )(pred_masks, tail, labels_t, pscore_row)
    return out[0, :K]


# speedup vs baseline: 1.3654x; 1.0016x over previous
import functools

import jax
import jax.numpy as jnp
from jax.experimental import pallas as pl
from jax.experimental.pallas import tpu as pltpu

Q = 512
S = 20000
C = 18
THR = 0.75
K = 100
TILE = 2048
SALIGN = (S // 128) * 128
NSTEP = (SALIGN + TILE - 1) // TILE
NDMA = 4


def _body(masks_hbm, tail_ref, labels_ref, pscore_ref, out_ref,
          buf0, buf1, sem):
    bufs = (buf0, buf1)
    widths = [TILE] * (NSTEP - 1) + [SALIGN - TILE * (NSTEP - 1)]
    RS = Q // NDMA

    def stripe(j, k):
        return pltpu.make_async_copy(
            masks_hbm.at[pl.ds(k * RS, RS), pl.ds(j * TILE, widths[j])],
            bufs[j % 2].at[pl.ds(k * RS, RS), pl.ds(0, widths[j])],
            sem.at[j % 2, k])

    def start_copy(j):
        for k in range(NDMA):
            stripe(j, k).start()

    def wait_copy(j):
        for k in range(NDMA):
            stripe(j, k).wait()

    start_copy(0)
    inter = jnp.zeros((Q, Q), jnp.float32)
    sigrow = jnp.zeros((1, Q), jnp.float32)

    def accum(x, inter, sigrow):
        w = x.shape[1]
        bin16 = jnp.where(x > 0.0, 1.0, 0.0).astype(jnp.bfloat16)
        inter = inter + jax.lax.dot_general(
            bin16, bin16, (((1,), (1,)), ((), ())),
            preferred_element_type=jnp.float32)
        th = jnp.maximum(jnp.tanh(0.5 * x), 0.0).astype(jnp.bfloat16)
        sigrow = sigrow + jax.lax.dot_general(
            jnp.ones((1, w), jnp.bfloat16), th, (((1,), (1,)), ((), ())),
            preferred_element_type=jnp.float32)
        return inter, sigrow

    for j in range(NSTEP):
        if j + 1 < NSTEP:
            start_copy(j + 1)
        wait_copy(j)
        x = bufs[j % 2][:, :widths[j]]
        inter, sigrow = accum(x, inter, sigrow)
    inter, sigrow = accum(tail_ref[...], inter, sigrow)

    lt = labels_ref[...]
    m = jnp.max(lt, axis=0, keepdims=True)
    e = jnp.exp(lt - m)
    denom = jnp.sum(e, axis=0, keepdims=True)
    scores_t = (e[:C, :] / denom) * pscore_ref[...]

    ri = jax.lax.broadcasted_iota(jnp.int32, (Q, Q), 0)
    ci = jax.lax.broadcasted_iota(jnp.int32, (Q, Q), 1)
    eye = ri == ci
    pn_row = jnp.sum(jnp.where(eye, inter, 0.0), axis=0, keepdims=True)
    pn_col = jnp.sum(jnp.where(eye, inter, 0.0), axis=1, keepdims=True)

    nms_row = jnp.max(scores_t, axis=0, keepdims=True)
    nms_row = jnp.where(pn_row == 0.0, 0.0, nms_row)

    union = pn_col + pn_row - inter + 1e-6
    adj = jnp.where(inter > THR * union, 1.0, 0.0).astype(jnp.bfloat16)

    s_col = jnp.sum(jnp.where(eye, jnp.broadcast_to(nms_row, (Q, Q)), 0.0),
                    axis=1, keepdims=True)
    gt = jnp.where(nms_row > s_col, 1.0, 0.0)
    eqlt = jnp.where((nms_row == s_col) & (ci < ri), 1.0, 0.0)
    rank_col = jnp.sum(gt + eqlt, axis=1, keepdims=True)
    rank_row = jnp.sum(jnp.where(eye, jnp.broadcast_to(rank_col, (Q, Q)), 0.0),
                       axis=0, keepdims=True)
    perm16 = jnp.where(ri.astype(jnp.float32) == rank_row, 1.0, 0.0
                       ).astype(jnp.bfloat16)

    padj = jax.lax.dot_general(
        perm16, adj, (((1,), (0,)), ((), ())),
        preferred_element_type=jnp.float32)
    sadj = jax.lax.dot_general(
        padj.astype(jnp.bfloat16), perm16, (((1,), (1,)), ((), ())),
        preferred_element_type=jnp.float32)
    sadj16 = jnp.where(ci > ri, sadj, 0.0).astype(jnp.bfloat16)

    def nms_cond(carry):
        _, changed = carry
        return changed

    def one_sweep(keep):
        supcnt = jax.lax.dot_general(
            keep.astype(jnp.bfloat16), sadj16, (((1,), (0,)), ((), ())),
            preferred_element_type=jnp.float32)
        return jnp.where(supcnt > 0.0, 0.0, 1.0)

    def nms_sweep(carry):
        keep, _ = carry
        keep_new = one_sweep(one_sweep(keep))
        changed = jnp.sum(jnp.abs(keep_new - keep)) > 0.0
        return keep_new, changed

    keep, _ = jax.lax.while_loop(
        nms_cond, nms_sweep, (jnp.ones((1, Q), jnp.float32), True))

    keep_row = jax.lax.dot_general(
        keep.astype(jnp.bfloat16), perm16, (((1,), (0,)), ((), ())),
        preferred_element_type=jnp.float32)

    w_row = (0.5 * pn_row + 0.5 * sigrow) / (pn_row + 1e-6)
    a = scores_t * keep_row

    wb = jnp.broadcast_to(w_row, (C, Q))
    lane128 = jax.lax.broadcasted_iota(jnp.int32, (1, 128), 1)

    acc = jnp.zeros((1, 128), jnp.float32)
    for k in range(K):
        v = jnp.max(a)
        hit = a == v
        wv = jnp.sum(jnp.where(hit, wb, 0.0))
        acc = jnp.where(lane128 == k, jnp.maximum(v, 0.0) * wv, acc)
        a = jnp.where(hit, -1.0, a)
    out_ref[...] = acc


@functools.partial(jax.jit, static_argnames=())
def kernel(pred_labels, pred_masks, pred_scores):
    labels_t = pred_labels.T
    pscore_row = pred_scores.reshape(1, Q)
    tail = jax.lax.slice(pred_masks, (0, SALIGN), (Q, S))
    out = pl.pallas_call(
        _body,
        in_specs=[
            pl.BlockSpec(memory_space=pltpu.MemorySpace.HBM),
            pl.BlockSpec((Q, S - SALIGN), lambda: (0, 0)),
            pl.BlockSpec((C + 1, Q), lambda: (0, 0)),
            pl.BlockSpec((1, Q), lambda: (0, 0)),
        ],
        out_specs=pl.BlockSpec((1, 128), lambda: (0, 0)),
        out_shape=jax.ShapeDtypeStruct((1, 128), jnp.float32),
        scratch_shapes=[
            pltpu.VMEM((Q, TILE), jnp.float32),
            pltpu.VMEM((Q, TILE), jnp.float32),
            pltpu.SemaphoreType.DMA((2, NDMA)),
        ],
    )(pred_masks, tail, labels_t, pscore_row)
    return out[0, :K]
